# baseline (speedup 1.0000x reference)
import jax
import jax.numpy as jnp
import numpy as np
from jax.experimental import pallas as pl
from jax.experimental.pallas import tpu as pltpu

_B, _S, _D, _E = 1, 2048, 768, 8
_N = _B * _S
_MAX_THRESHOLD = 0.1
_GCOLS = 16


def _moe_body(x_ref, wg_ref, bias_ref, eb_ref, ew_ref, out_ref, wt_scr):
    e = pl.program_id(0)
    acc = jnp.dot(x_ref[...], ew_ref[0], preferred_element_type=jnp.float32)

    @pl.when(e == 0)
    def _routing():
        logits = jnp.dot(x_ref[...], wg_ref[...],
                         preferred_element_type=jnp.float32) + bias_ref[...]
        lt = logits.T
        g = lt[:_E, :]
        g = g - jnp.max(g, axis=0, keepdims=True)
        g = jnp.exp(g)
        g = g / jnp.sum(g, axis=0, keepdims=True)
        thr = jax.nn.sigmoid(lt[_E:_E + 1, :]) * _MAX_THRESHOLD
        ad = g - thr
        w = jnp.where(ad >= 0.0, ad, 0.0)
        s = jnp.sum(w, axis=0, keepdims=True)
        w = w / jnp.where(s == 0.0, 1.0, s)
        wt_scr[...] = w

    wcol = wt_scr[pl.ds(e, 1), :].T

    @pl.when(e == 0)
    def _init():
        out_ref[...] = wcol * acc + jax.lax.dot_general(
            wt_scr[...], eb_ref[...], (((0,), (0,)), ((), ())),
            preferred_element_type=jnp.float32)

    @pl.when(e > 0)
    def _accum():
        out_ref[...] += wcol * acc


def kernel(inputs, gate_W, gate_b, thr_W, thr_b, exp_W, exp_b):
    flat = inputs.reshape(_N, _D)
    wg = jnp.zeros((_D, _GCOLS), dtype=jnp.float32)
    wg = wg.at[:, :_E].set(gate_W).at[:, _E:_E + 1].set(thr_W)
    bias = jnp.zeros((1, _GCOLS), dtype=jnp.float32)
    bias = bias.at[:, :_E].set(gate_b[None, :]).at[:, _E].set(thr_b[0])

    out = pl.pallas_call(
        _moe_body,
        grid=(_E,),
        in_specs=[
            pl.BlockSpec((_N, _D), lambda e: (0, 0)),
            pl.BlockSpec((_D, _GCOLS), lambda e: (0, 0)),
            pl.BlockSpec((1, _GCOLS), lambda e: (0, 0)),
            pl.BlockSpec((_E, _D), lambda e: (0, 0)),
            pl.BlockSpec((1, _D, _D), lambda e: (e, 0, 0)),
        ],
        out_specs=pl.BlockSpec((_N, _D), lambda e: (0, 0)),
        out_shape=jax.ShapeDtypeStruct((_N, _D), jnp.float32),
        scratch_shapes=[pltpu.VMEM((_E, _N), jnp.float32)],
        compiler_params=pltpu.CompilerParams(
            dimension_semantics=("arbitrary",),
        ),
    )(flat, wg, bias, exp_b, exp_W)
    return out.reshape(inputs.shape[:-1] + (_D,))


# speedup vs baseline: 1.1313x; 1.1313x over previous
import jax
import jax.numpy as jnp
import numpy as np
from jax.experimental import pallas as pl
from jax.experimental.pallas import tpu as pltpu

_B, _S, _D, _E = 1, 2048, 768, 8
_N = _B * _S
_MAX_THRESHOLD = 0.1
_GCOLS = 16


def _moe_body(x_ref, wg_ref, bias_ref, eb_ref, ew_ref, out_ref, wt_scr):
    s = pl.program_id(0)

    @pl.when(s == 0)
    def _routing():
        logits = jnp.dot(x_ref[...], wg_ref[...],
                         preferred_element_type=jnp.float32) + bias_ref[...]
        lt = logits.T
        g = lt[:_E, :]
        g = g - jnp.max(g, axis=0, keepdims=True)
        g = jnp.exp(g)
        g = g / jnp.sum(g, axis=0, keepdims=True)
        thr = jax.nn.sigmoid(lt[_E:_E + 1, :]) * _MAX_THRESHOLD
        ad = g - thr
        w = jnp.where(ad >= 0.0, ad, 0.0)
        sw = jnp.sum(w, axis=0, keepdims=True)
        w = w / jnp.where(sw == 0.0, 1.0, sw)
        wt_scr[...] = w
        out_ref[...] = jax.lax.dot_general(
            w, eb_ref[...], (((0,), (0,)), ((), ())),
            preferred_element_type=jnp.float32)

    @pl.when(s > 0)
    def _expert():
        acc = jnp.dot(x_ref[...], ew_ref[0],
                      preferred_element_type=jnp.float32)
        wcol = wt_scr[pl.ds(s - 1, 1), :].T
        out_ref[...] += wcol * acc


def kernel(inputs, gate_W, gate_b, thr_W, thr_b, exp_W, exp_b):
    flat = inputs.reshape(_N, _D)
    wg = jnp.zeros((_D, _GCOLS), dtype=jnp.float32)
    wg = wg.at[:, :_E].set(gate_W).at[:, _E:_E + 1].set(thr_W)
    bias = jnp.zeros((1, _GCOLS), dtype=jnp.float32)
    bias = bias.at[:, :_E].set(gate_b[None, :]).at[:, _E].set(thr_b[0])

    out = pl.pallas_call(
        _moe_body,
        grid=(_E + 1,),
        in_specs=[
            pl.BlockSpec((_N, _D), lambda s: (0, 0)),
            pl.BlockSpec((_D, _GCOLS), lambda s: (0, 0)),
            pl.BlockSpec((1, _GCOLS), lambda s: (0, 0)),
            pl.BlockSpec((_E, _D), lambda s: (0, 0)),
            pl.BlockSpec((1, _D, _D), lambda s: (jnp.maximum(s - 1, 0), 0, 0)),
        ],
        out_specs=pl.BlockSpec((_N, _D), lambda s: (0, 0)),
        out_shape=jax.ShapeDtypeStruct((_N, _D), jnp.float32),
        scratch_shapes=[pltpu.VMEM((_E, _N), jnp.float32)],
        compiler_params=pltpu.CompilerParams(
            dimension_semantics=("arbitrary",),
        ),
    )(flat, wg, bias, exp_b, exp_W)
    return out.reshape(inputs.shape[:-1] + (_D,))
